# baseline (speedup 1.0000x reference)
import functools

import jax
import jax.numpy as jnp
from jax import lax
from jax.experimental import pallas as pl
from jax.experimental.pallas import tpu as pltpu
from jax.experimental.pallas import tpu_sc as plsc

_D = 5
_DP = 8
_B = 16384
_H = 200
_TOTAL = _B * _H
_NW = 32
_NT_H = _H // 8
_NT_B = _B // 128
_TPW = _NT_B // _NW
_CL = _TPW * 8 * 128
_ROW_W = _NT_B * 8 * 128
_PLANE = _NT_H * _ROW_W

_mesh = plsc.VectorSubcoreMesh(core_axis_name="c", subcore_axis_name="s")

_V = 1000000
_ROWS_W = _V // _NW
_CK = 3125
_NCK = _ROWS_W // _CK


@functools.partial(
    pl.kernel,
    mesh=_mesh,
    out_type=jax.ShapeDtypeStruct((_V, _DP), jnp.float32),
    scratch_types=[pltpu.VMEM((_CK, _D), jnp.float32)],
    compiler_params=pltpu.CompilerParams(
        use_tc_tiling_on_sc=False, needs_layout_passes=False
    ),
)
def _pad_table(t_hbm, out_hbm, v5):
    wid = lax.axis_index("s") * 2 + lax.axis_index("c")

    def body(i, c):
        r0 = wid * _ROWS_W + i * _CK
        pltpu.sync_copy(t_hbm.at[pl.ds(r0, _CK)], v5)
        pltpu.sync_copy(v5, out_hbm.at[pl.ds(r0, _CK), pl.ds(0, _D)])
        return c

    lax.fori_loop(0, _NCK, body, 0)


@functools.partial(
    pl.kernel,
    mesh=_mesh,
    out_type=jax.ShapeDtypeStruct((_D * _TOTAL,), jnp.float32),
    scratch_types=[
        pltpu.VMEM((_CL,), jnp.int32),
        pltpu.VMEM((_CL, _DP), jnp.float32),
        pltpu.VMEM((_D, _CL), jnp.float32),
        pltpu.SemaphoreType.DMA,
    ],
    compiler_params=pltpu.CompilerParams(
        use_tc_tiling_on_sc=False, needs_layout_passes=False
    ),
)
def _embed_gather_t(idx_hbm, table_hbm, out_hbm, idx_v, rows_v, planes_v, sem):
    wid = lax.axis_index("s") * 2 + lax.axis_index("c")
    lane = lax.iota(jnp.int32, 16)
    cols = [jnp.full((16,), d, jnp.int32) for d in range(_D)]

    def body(ht, carry):
        src = ht * _ROW_W + wid * _CL
        pltpu.sync_copy(idx_hbm.at[pl.ds(src, _CL)], idx_v)
        pltpu.async_copy(table_hbm.at[idx_v], rows_v, sem).wait()

        def tbody(j, c):
            row16 = j * 16 + lane
            for d in range(_D):
                v = plsc.load_gather(rows_v, [row16, cols[d]])
                planes_v[d, pl.ds(j * 16, 16)] = v
            return c

        lax.fori_loop(0, _CL // 16, tbody, 0)
        for d in range(_D):
            dst = d * _PLANE + ht * _ROW_W + wid * _CL
            pltpu.sync_copy(planes_v.at[d], out_hbm.at[pl.ds(dst, _CL)])
        return carry

    lax.fori_loop(0, _NT_H, body, 0)


def kernel(data_index, embedding_dim, table5):
    flag = jnp.asarray(embedding_dim == _D, jnp.int32)
    idx_t = (
        data_index.T.reshape(_NT_H, 8, _NT_B, 128)
        .transpose(0, 2, 1, 3)
        .reshape(_TOTAL)
    ) * flag
    table8 = _pad_table(table5)
    flat = _embed_gather_t(idx_t, table8)
    o5 = flat.reshape(_D, _NT_H, _NT_B, 8, 128).transpose(1, 3, 2, 4, 0)
    return o5.reshape(_H, _B, _D).transpose(1, 0, 2)


# speedup vs baseline: 2.7757x; 2.7757x over previous
import functools

import jax
import jax.numpy as jnp
from jax import lax
from jax.experimental import pallas as pl
from jax.experimental.pallas import tpu as pltpu
from jax.experimental.pallas import tpu_sc as plsc

_D = 5
_B = 16384
_H = 200
_TOTAL = _B * _H
_NW = 32
_V = 1000000
_RW = 40
_VR = _V * _D // _RW
_NT_H = _H // 8
_NT_B = _B // 128
_CL = 2048
_CPH = _NT_B * 1024 // (_CL * _NW)
_ROW_W = _NT_B * 1024
_PLANE = _NT_H * _ROW_W

_mesh = plsc.VectorSubcoreMesh(core_axis_name="c", subcore_axis_name="s")


@functools.partial(
    pl.kernel,
    mesh=_mesh,
    out_type=jax.ShapeDtypeStruct((_D * _TOTAL,), jnp.float32),
    scratch_types=[
        pltpu.VMEM((_CL,), jnp.int32),
        pltpu.VMEM((_CL,), jnp.int32),
        pltpu.VMEM((_CL, _RW), jnp.float32),
        pltpu.VMEM((_D, _CL), jnp.float32),
        pltpu.SemaphoreType.DMA,
    ],
    compiler_params=pltpu.CompilerParams(
        use_tc_tiling_on_sc=False, needs_layout_passes=False
    ),
)
def _embed_gather_t(idx_hbm, table_hbm, out_hbm, idx_v, row_v, rows_v, planes_v,
                    sem):
    wid = lax.axis_index("s") * 2 + lax.axis_index("c")
    lane = lax.iota(jnp.int32, 16)

    def body(i, carry):
        ht = i // _CPH
        half = i % _CPH
        off = ht * _ROW_W + (wid * _CPH + half) * _CL
        pltpu.sync_copy(idx_hbm.at[pl.ds(off, _CL)], idx_v)

        def rbody(j, c):
            w16 = idx_v[pl.ds(j * 16, 16)]
            row_v[pl.ds(j * 16, 16)] = lax.shift_right_logical(w16, 3)
            return c

        lax.fori_loop(0, _CL // 16, rbody, 0)
        pltpu.async_copy(table_hbm.at[row_v], rows_v, sem).wait()

        def tbody(j, c):
            w16 = idx_v[pl.ds(j * 16, 16)]
            col = jnp.bitwise_and(w16, 7) * _D
            row16 = j * 16 + lane
            for d in range(_D):
                v = plsc.load_gather(rows_v, [row16, col + d])
                planes_v[d, pl.ds(j * 16, 16)] = v
            return c

        lax.fori_loop(0, _CL // 16, tbody, 0)
        for d in range(_D):
            dst = d * _PLANE + off
            pltpu.sync_copy(planes_v.at[d], out_hbm.at[pl.ds(dst, _CL)])
        return carry

    lax.fori_loop(0, _NT_H * _CPH, body, 0)


def kernel(data_index, embedding_dim, table5):
    flag = jnp.asarray(embedding_dim == _D, jnp.int32)
    idx_t = (
        data_index.T.reshape(_NT_H, 8, _NT_B, 128)
        .transpose(0, 2, 1, 3)
        .reshape(_TOTAL)
    ) * flag
    table40 = table5.reshape(_VR, _RW)
    flat = _embed_gather_t(idx_t, table40)
    o5 = flat.reshape(_D, _NT_H, _NT_B, 8, 128).transpose(1, 3, 2, 4, 0)
    return o5.reshape(_H, _B, _D).transpose(1, 0, 2)


# speedup vs baseline: 3.1556x; 1.1369x over previous
import functools

import jax
import jax.numpy as jnp
from jax import lax
from jax.experimental import pallas as pl
from jax.experimental.pallas import tpu as pltpu
from jax.experimental.pallas import tpu_sc as plsc

_D = 5
_B = 16384
_H = 200
_TOTAL = _B * _H
_NW = 32
_V = 1000000
_RW = 40
_VR = _V * _D // _RW
_NT_H = _H // 8
_NT_B = _B // 128
_CL = 1024
_CPH = _NT_B * 1024 // (_CL * _NW)
_ROW_W = _NT_B * 1024
_PLANE = _NT_H * _ROW_W
_NCH = _NT_H * _CPH

_mesh = plsc.VectorSubcoreMesh(core_axis_name="c", subcore_axis_name="s")


@functools.partial(
    pl.kernel,
    mesh=_mesh,
    out_type=jax.ShapeDtypeStruct((_D * _TOTAL,), jnp.float32),
    scratch_types=[
        pltpu.VMEM((2, _CL), jnp.int32),
        pltpu.VMEM((2, _CL), jnp.int32),
        pltpu.VMEM((2, _CL, _RW), jnp.float32),
        pltpu.VMEM((2 * _D, _CL), jnp.float32),
        pltpu.SemaphoreType.DMA,
        pltpu.SemaphoreType.DMA,
    ],
    compiler_params=pltpu.CompilerParams(
        use_tc_tiling_on_sc=False, needs_layout_passes=False
    ),
)
def _embed_gather_t(idx_hbm, table_hbm, out_hbm, idx_v, row_v, rows_v, planes_v,
                    sem_g, sem_w):
    wid = lax.axis_index("s") * 2 + lax.axis_index("c")
    lane = lax.iota(jnp.int32, 16)

    def chunk_off(i):
        ht = i // _CPH
        q = i % _CPH
        return ht * _ROW_W + (wid * _CPH + q) * _CL

    def start_gather(i, b):
        off = chunk_off(i)
        pltpu.sync_copy(idx_hbm.at[pl.ds(off, _CL)], idx_v.at[b])

        def rbody(j, c):
            w16 = idx_v[b, pl.ds(j * 16, 16)]
            row_v[b, pl.ds(j * 16, 16)] = lax.shift_right_logical(w16, 3)
            return c

        lax.fori_loop(0, _CL // 16, rbody, 0)
        return pltpu.async_copy(table_hbm.at[row_v.at[b]], rows_v.at[b], sem_g)

    def extract(i, b):
        bb = jnp.full((16,), b, jnp.int32)

        def tbody(j, c):
            w16 = idx_v[b, pl.ds(j * 16, 16)]
            col = jnp.bitwise_and(w16, 7) * _D
            row16 = j * 16 + lane
            for d in range(_D):
                v = plsc.load_gather(rows_v, [bb, row16, col + d])
                planes_v[b * _D + d, pl.ds(j * 16, 16)] = v
            return c

        lax.fori_loop(0, _CL // 16, tbody, 0)
        off = chunk_off(i)
        return [
            pltpu.async_copy(
                planes_v.at[b * _D + d],
                out_hbm.at[pl.ds(d * _PLANE + off, _CL)],
                sem_w,
            )
            for d in range(_D)
        ]

    def body(p, carry):
        c0 = start_gather(2 * p, 0)
        c1 = start_gather(2 * p + 1, 1)
        c0.wait()
        w0 = extract(2 * p, 0)
        c1.wait()
        w1 = extract(2 * p + 1, 1)
        for w in w0 + w1:
            w.wait()
        return carry

    lax.fori_loop(0, _NCH // 2, body, 0)


def kernel(data_index, embedding_dim, table5):
    flag = jnp.asarray(embedding_dim == _D, jnp.int32)
    idx_t = (
        data_index.T.reshape(_NT_H, 8, _NT_B, 128)
        .transpose(0, 2, 1, 3)
        .reshape(_TOTAL)
    ) * flag
    table40 = table5.reshape(_VR, _RW)
    flat = _embed_gather_t(idx_t, table40)
    o5 = flat.reshape(_D, _NT_H, _NT_B, 8, 128).transpose(1, 3, 2, 4, 0)
    return o5.reshape(_H, _B, _D).transpose(1, 0, 2)
